# baseline (speedup 1.0000x reference)
import jax
import jax.numpy as jnp
from jax import lax
from jax.experimental import pallas as pl
from jax.experimental.pallas import tpu as pltpu
from jax.experimental.pallas import tpu_sc as plsc

M = 524288
D = 64
B = 65536

NC = 2
NS = 16
L = 16

M2 = M // 2
SLAB_P = 8192
STRIPE_P = SLAB_P // NS
HALF_P = M2 // NC
NSLAB = HALF_P // SLAB_P
SHARE = B // NS
CH = 64


def _body(mem_hbm, val_hbm, idx_hbm, out_hbm,
          shared, idx_v, mb_v, ml_v, rows_v, sem):
    c = lax.axis_index("c")
    s = lax.axis_index("s")
    share_base = s * SHARE
    half_base = c * HALF_P
    lane = lax.broadcasted_iota(jnp.int32, (L,), 0)

    pltpu.sync_copy(idx_hbm.at[pl.ds(share_base, SHARE)], idx_v)

    def slab_body(si, carry):
        base = half_base + si * SLAB_P
        pltpu.sync_copy(mem_hbm.at[pl.ds(base + s * STRIPE_P, STRIPE_P)],
                        shared.at[pl.ds(s * STRIPE_P, STRIPE_P)])
        plsc.subcore_barrier()

        def scan_body(i, cnt):
            iv = idx_v[pl.ds(i * L, L)]
            pv = iv >> 1
            m = (pv >= base) & (pv < base + SLAB_P)
            cum = plsc.cumsum(jnp.where(m, 1, 0).astype(jnp.int32))
            pos = cnt + cum - 1
            bv = share_base + i * L + lane
            plsc.store_scatter(mb_v, [pos >> 6, pos & 63],
                               (bv << 1) | (iv & 1), mask=m)
            plsc.store_scatter(ml_v, [pos >> 6, pos & 63], pv - base,
                               mask=m)
            return cnt + plsc.all_reduce_population_count(m)[0]

        cnt = lax.fori_loop(0, SHARE // L, scan_body, jnp.int32(0))

        padt = jnp.full((L,), SLAB_P, jnp.int32) + s
        padb = jnp.zeros((L,), jnp.int32)
        for k in range(CH // L):
            p = cnt + k * L + lane
            plsc.store_scatter(mb_v, [p >> 6, p & 63], padb)
            plsc.store_scatter(ml_v, [p >> 6, p & 63], padt)

        def chunk_body(j, carry2):
            pltpu.async_copy(val_hbm.at[mb_v.at[j]], rows_v, sem).wait()
            pltpu.sync_copy(rows_v, shared.at[ml_v.at[j]], add=True)
            return carry2

        nch = (cnt + (CH - 1)) >> 6
        lax.fori_loop(0, nch, chunk_body, jnp.int32(0))
        plsc.subcore_barrier()

        pltpu.sync_copy(shared.at[pl.ds(s * STRIPE_P, STRIPE_P)],
                        out_hbm.at[pl.ds(base + s * STRIPE_P, STRIPE_P)])
        plsc.subcore_barrier()
        return carry

    lax.fori_loop(0, NSLAB, slab_body, jnp.int32(0))


_sc_scatter_add = pl.kernel(
    _body,
    out_type=jax.ShapeDtypeStruct((M2, 128), jnp.float32),
    mesh=plsc.VectorSubcoreMesh(core_axis_name="c", subcore_axis_name="s"),
    compiler_params=pltpu.CompilerParams(needs_layout_passes=False),
    scratch_types=[
        pltpu.VMEM_SHARED((SLAB_P + NS, 128), jnp.float32),
        pltpu.VMEM((SHARE,), jnp.int32),
        pltpu.VMEM(((SHARE + CH) // CH, CH), jnp.int32),
        pltpu.VMEM(((SHARE + CH) // CH, CH), jnp.int32),
        pltpu.VMEM((CH, 128), jnp.float32),
        pltpu.SemaphoreType.DMA,
    ],
)


def kernel(mem, val, idx):
    mem_p = mem.reshape(M2, 128)
    zeros = jnp.zeros_like(val)
    val_lr = jnp.stack(
        [jnp.concatenate([val, zeros], axis=1),
         jnp.concatenate([zeros, val], axis=1)], axis=1,
    ).reshape(2 * B, 128)
    out_p = _sc_scatter_add(mem_p, val_lr, idx)
    return out_p.reshape(M, D)
